# Initial kernel scaffold; baseline (speedup 1.0000x reference)
import functools

import jax
import jax.numpy as jnp
from jax import lax
from jax.experimental import pallas as pl
from jax.experimental.pallas import tpu as pltpu
from jax.experimental.pallas import tpu_sc as plsc

VOCAB = 32
HIDDEN = 128
B = 4096
L = 200
N = B * L
NC = 2
NS = 16
NW = NC * NS
PER_W = N // NW
LANES = 16


def _table_body(emb_ref, w1_ref, b1_ref, w2_ref, b2_ref, out_ref):
    h = jnp.dot(emb_ref[...], w1_ref[...], preferred_element_type=jnp.float32)
    h = jnp.maximum(h + b1_ref[...], 0.0)
    v = jnp.sum(h * w2_ref[...], axis=1, keepdims=True) + b2_ref[...]
    out_ref[...] = v


def _compute_table(emb, W1, b1, W2, b2):
    return pl.pallas_call(
        _table_body,
        out_shape=jax.ShapeDtypeStruct((VOCAB, 1), jnp.float32),
    )(emb, W1, b1.reshape(1, HIDDEN), W2.reshape(1, HIDDEN), b2.reshape(1, 1))


_MESH = plsc.VectorSubcoreMesh(core_axis_name="c", subcore_axis_name="s")


@functools.partial(
    pl.kernel,
    mesh=_MESH,
    out_type=jax.ShapeDtypeStruct((N,), jnp.float32),
    scratch_types=[
        pltpu.VMEM((VOCAB,), jnp.float32),
        pltpu.VMEM((PER_W,), jnp.int32),
        pltpu.VMEM((PER_W,), jnp.float32),
    ],
)
def _gather_kernel(table_hbm, x_hbm, out_hbm, table_v, idx_v, out_v):
    wid = lax.axis_index("s") * NC + lax.axis_index("c")
    base = wid * PER_W
    pltpu.sync_copy(table_hbm, table_v)
    pltpu.sync_copy(x_hbm.at[pl.ds(base, PER_W)], idx_v)

    def body(i, carry):
        off = i * LANES
        idx16 = idx_v[pl.ds(off, LANES)]
        out_v[pl.ds(off, LANES)] = plsc.load_gather(table_v, [idx16])
        return carry

    lax.fori_loop(0, PER_W // LANES, body, 0)
    pltpu.sync_copy(out_v, out_hbm.at[pl.ds(base, PER_W)])


def kernel(x, emb, W1, b1, W2, b2):
    table = _compute_table(emb, W1, b1, W2, b2).reshape(VOCAB)
    out_flat = _gather_kernel(table, x.reshape(N))
    return out_flat.reshape(B, L, 1)


# speedup vs baseline: 54.3706x; 54.3706x over previous
import functools

import jax
import jax.numpy as jnp
from jax import lax
from jax.experimental import pallas as pl
from jax.experimental.pallas import tpu as pltpu
from jax.experimental.pallas import tpu_sc as plsc

VOCAB = 32
HIDDEN = 128
B = 4096
L = 200
N = B * L
NC = 2
NS = 16
NW = NC * NS
PER_W = N // NW
LANES = 16


def _table_body(emb_ref, w1_ref, b1_ref, w2_ref, b2_ref, out_ref):
    h = jnp.dot(emb_ref[...], w1_ref[...], preferred_element_type=jnp.float32)
    h = jnp.maximum(h + b1_ref[...], 0.0)
    v = jnp.sum(h * w2_ref[...], axis=1, keepdims=True) + b2_ref[...]
    out_ref[...] = v


def _compute_table(emb, W1, b1, W2, b2):
    return pl.pallas_call(
        _table_body,
        out_shape=jax.ShapeDtypeStruct((VOCAB, 1), jnp.float32),
    )(emb, W1, b1.reshape(1, HIDDEN), W2.reshape(1, HIDDEN), b2.reshape(1, 1))


_MESH = plsc.VectorSubcoreMesh(core_axis_name="c", subcore_axis_name="s")

_GATHER_DNUMS = lax.GatherDimensionNumbers(
    offset_dims=(), collapsed_slice_dims=(0,), start_index_map=(0,))


def _vreg_gather(src16, idx16):
    return lax.gather(
        src16, idx16[:, None], dimension_numbers=_GATHER_DNUMS,
        slice_sizes=(1,), mode=lax.GatherScatterMode.PROMISE_IN_BOUNDS)


@functools.partial(
    pl.kernel,
    mesh=_MESH,
    out_type=jax.ShapeDtypeStruct((N,), jnp.float32),
    scratch_types=[
        pltpu.VMEM((VOCAB,), jnp.float32),
        pltpu.VMEM((PER_W,), jnp.int32),
        pltpu.VMEM((PER_W,), jnp.float32),
    ],
)
def _gather_kernel(table_hbm, x_hbm, out_hbm, table_v, idx_v, out_v):
    wid = lax.axis_index("s") * NC + lax.axis_index("c")
    base = wid * PER_W
    pltpu.sync_copy(table_hbm, table_v)
    pltpu.sync_copy(x_hbm.at[pl.ds(base, PER_W)], idx_v)

    t_lo = table_v[pl.ds(0, LANES)]
    t_hi = table_v[pl.ds(LANES, LANES)]

    def lookup(idx16):
        low = jnp.bitwise_and(idx16, LANES - 1)
        g_lo = _vreg_gather(t_lo, low)
        g_hi = _vreg_gather(t_hi, low)
        return jnp.where(idx16 < LANES, g_lo, g_hi)

    def body(i, carry):
        off = i * LANES
        out_v[pl.ds(off, LANES)] = lookup(idx_v[pl.ds(off, LANES)])
        return carry

    lax.fori_loop(0, PER_W // LANES, body, 0)
    pltpu.sync_copy(out_v, out_hbm.at[pl.ds(base, PER_W)])


def kernel(x, emb, W1, b1, W2, b2):
    table = _compute_table(emb, W1, b1, W2, b2).reshape(VOCAB)
    out_flat = _gather_kernel(table, x.reshape(N))
    return out_flat.reshape(B, L, 1)


# speedup vs baseline: 125.5661x; 2.3094x over previous
import functools

import jax
import jax.numpy as jnp
from jax import lax
from jax.experimental import pallas as pl
from jax.experimental.pallas import tpu as pltpu
from jax.experimental.pallas import tpu_sc as plsc

VOCAB = 32
HIDDEN = 128
B = 4096
L = 200
N = B * L
NC = 2
NS = 16
NW = NC * NS
PER_W = N // NW
LANES = 16
RB = L // 8
CB = B // 128


def _table_body(emb_ref, w1_ref, b1_ref, w2_ref, b2_ref, out_ref):
    h = jnp.dot(emb_ref[...], w1_ref[...], preferred_element_type=jnp.float32)
    h = jnp.maximum(h + b1_ref[...], 0.0)
    v = lax.dot_general(w2_ref[...], h, (((1,), (1,)), ((), ())),
                        preferred_element_type=jnp.float32)
    out_ref[:, :VOCAB] = v + b2_ref[...]


def _compute_table(emb, W1, b1, W2, b2):
    return pl.pallas_call(
        _table_body,
        out_shape=jax.ShapeDtypeStruct((1, 128), jnp.float32),
    )(emb, W1, b1.reshape(1, HIDDEN), W2.reshape(1, HIDDEN), b2.reshape(1, 1))


_MESH = plsc.VectorSubcoreMesh(core_axis_name="c", subcore_axis_name="s")

_GATHER_DNUMS = lax.GatherDimensionNumbers(
    offset_dims=(), collapsed_slice_dims=(0,), start_index_map=(0,))


def _vreg_gather(src16, idx16):
    return lax.gather(
        src16, idx16[:, None], dimension_numbers=_GATHER_DNUMS,
        slice_sizes=(1,), mode=lax.GatherScatterMode.PROMISE_IN_BOUNDS)


@functools.partial(
    pl.kernel,
    mesh=_MESH,
    out_type=jax.ShapeDtypeStruct((L, CB, 128), jnp.float32),
    scratch_types=[
        pltpu.VMEM((128,), jnp.float32),
        pltpu.VMEM((RB, 8, 128), jnp.int32),
        pltpu.VMEM((L, 128), jnp.float32),
        pltpu.SemaphoreType.DMA,
        pltpu.SemaphoreType.DMA,
    ],
)
def _gather_kernel(table_hbm, x_hbm, out_hbm, table_v, idx_v, out_v,
                   sem_in, sem_out):
    w = lax.axis_index("s") * NC + lax.axis_index("c")
    in_dma = pltpu.make_async_copy(x_hbm.at[:, w], idx_v, sem_in)
    in_dma.start()
    pltpu.sync_copy(table_hbm, table_v)
    t_lo = table_v[pl.ds(0, LANES)]
    t_hi = table_v[pl.ds(LANES, LANES)]
    in_dma.wait()

    def lookup(idx16):
        low = jnp.bitwise_and(idx16, LANES - 1)
        g_lo = _vreg_gather(t_lo, low)
        g_hi = _vreg_gather(t_hi, low)
        return jnp.where(idx16 < LANES, g_lo, g_hi)

    @plsc.parallel_loop(0, RB)
    def body(p):
        for r in range(8):
            for c8 in range(8):
                sl = pl.ds(c8 * LANES, LANES)
                out_v[p * 8 + r, sl] = lookup(idx_v[p, r, sl])
        pltpu.make_async_copy(
            out_v.at[pl.ds(p * 8, 8)],
            out_hbm.at[pl.ds(p * 8, 8), w], sem_out).start()
    pltpu.make_async_copy(out_v, out_hbm.at[:, w], sem_out).wait()


def kernel(x, emb, W1, b1, W2, b2):
    table = _compute_table(emb, W1, b1, W2, b2).reshape(128)
    xs = x.reshape(CB, 128, RB, 8).transpose(2, 0, 3, 1)
    out3 = _gather_kernel(table, xs)
    return out3.transpose(1, 2, 0).reshape(B, L, 1)
